# baseline (speedup 1.0000x reference)
import functools

import jax
import jax.numpy as jnp
from jax import lax
from jax.experimental import pallas as pl
from jax.experimental.pallas import tpu as pltpu
from jax.experimental.pallas import tpu_sc as plsc

EMBED_DIM = 64
NUM_NEG = 200
N_ROWS = 208
N_GROUPS = N_ROWS // 16
VALID_LAST = 9


def _softplus(z):
    e = jnp.exp(-jnp.abs(z))
    t = e / (e + 2.0)
    t2 = t * t
    ln_y = 2.0 * t * (1.0 + t2 * (1.0 / 3.0 + t2 * (1.0 / 5.0 + t2 * (1.0 / 7.0 + t2 * (1.0 / 9.0)))))
    return jnp.maximum(z, 0.0) + ln_y


def _sc_kernel(input_hbm, output_hbm, idx_c_hbm, idx_pn_hbm, out_hbm,
               idx_c_v, idx_pn_v, rows_c_v, rows_v, out_v, sem):
    is_lead = jnp.logical_and(lax.axis_index("c") == 0, lax.axis_index("s") == 0)

    @pl.when(is_lead)
    def _():
        pltpu.sync_copy(idx_c_hbm, idx_c_v)
        pltpu.sync_copy(idx_pn_hbm, idx_pn_v)

        cp0 = pltpu.make_async_copy(input_hbm.at[idx_c_v], rows_c_v, sem)
        cp1 = pltpu.make_async_copy(
            output_hbm.at[idx_pn_v.at[pl.ds(0, 104)]], rows_v.at[pl.ds(0, 104)], sem)
        cp2 = pltpu.make_async_copy(
            output_hbm.at[idx_pn_v.at[pl.ds(104, 104)]], rows_v.at[pl.ds(104, 104)], sem)
        cp0.start()
        cp1.start()
        cp2.start()
        cp0.wait()
        cp1.wait()
        cp2.wait()

        lanes = lax.iota(jnp.int32, 16)
        row_ids = [lanes + (16 * g) for g in range(N_GROUPS)]

        zero_ids = jnp.zeros((16,), jnp.int32)

        def body(j, accs):
            colj = jnp.full((16,), j, dtype=jnp.int32)
            cej = plsc.load_gather(rows_c_v, [zero_ids, colj])
            return tuple(
                acc + cej * plsc.load_gather(rows_v, [row_ids[g], colj])
                for g, acc in enumerate(accs)
            )

        zero = jnp.zeros((16,), jnp.float32)
        accs = lax.fori_loop(0, EMBED_DIM, body, (zero,) * N_GROUPS)

        total = zero
        for g in range(N_GROUPS):
            d = accs[g]
            if g == 0:
                d = jnp.where(lanes == 0, -d, d)
            if g == N_GROUPS - 1:
                d = jnp.where(lanes < VALID_LAST, d, -1e30)
            total = total + _softplus(d)

        out_v[...] = jnp.full((16,), jnp.sum(total))
        pltpu.sync_copy(out_v, out_hbm)


@jax.jit
def _run(center_word, positive_words, negative_words, input_embeddings, output_embeddings):
    idx_c = jnp.broadcast_to(center_word.astype(jnp.int32), (16,))
    idx_pn = jnp.concatenate([
        positive_words.astype(jnp.int32),
        negative_words.astype(jnp.int32),
        jnp.zeros((N_ROWS - 1 - NUM_NEG,), jnp.int32),
    ])
    mesh = plsc.VectorSubcoreMesh(core_axis_name="c", subcore_axis_name="s")
    k = functools.partial(
        pl.kernel,
        mesh=mesh,
        compiler_params=pltpu.CompilerParams(
            use_tc_tiling_on_sc=False, needs_layout_passes=False),
        out_type=jax.ShapeDtypeStruct((16,), jnp.float32),
        scratch_types=[
            pltpu.VMEM((16,), jnp.int32),
            pltpu.VMEM((N_ROWS,), jnp.int32),
            pltpu.VMEM((16, EMBED_DIM), jnp.float32),
            pltpu.VMEM((N_ROWS, EMBED_DIM), jnp.float32),
            pltpu.VMEM((16,), jnp.float32),
            pltpu.SemaphoreType.DMA,
        ],
    )(_sc_kernel)
    res = k(input_embeddings, output_embeddings, idx_c, idx_pn)
    return res[0].reshape(1, 1)


def kernel(center_word, positive_words, negative_words, input_embeddings, output_embeddings):
    return _run(center_word, positive_words, negative_words,
                input_embeddings, output_embeddings)
